# baseline (speedup 1.0000x reference)
import functools

import jax
import jax.numpy as jnp
from jax import lax
from jax.experimental import pallas as pl
from jax.experimental.pallas import tpu as pltpu
from jax.experimental.pallas import tpu_sc as plsc

DIM = 128
EPS = 1e-07

_NC = 2
_NS = 16
_NW = _NC * _NS
_CHUNK = 128
_LANES = 16
_W = DIM // 2


def _lane_gather(v, idx):
    dn = lax.GatherDimensionNumbers(
        offset_dims=(), collapsed_slice_dims=(0,), start_index_map=(0,))
    return lax.gather(v, idx[:, None], dn, slice_sizes=(1,),
                      mode=lax.GatherScatterMode.PROMISE_IN_BOUNDS)


def _sc_gather_bf16(ids_flat, word_table):
    n_tok = ids_flat.shape[0]
    assert n_tok % (_NW * _CHUNK) == 0
    per_w = n_tok // _NW
    n_chunks = per_w // _CHUNK
    n_super = (n_chunks - 2) // 3
    assert n_chunks == 3 * n_super + 2

    mesh = plsc.VectorSubcoreMesh(core_axis_name="c", subcore_axis_name="s")

    @functools.partial(
        pl.kernel,
        out_type=jax.ShapeDtypeStruct((n_tok, _W), jnp.int32),
        mesh=mesh,
        scratch_types=[
            pltpu.VMEM((per_w,), jnp.int32),
            pltpu.VMEM((_CHUNK, DIM), jnp.float32),
            pltpu.VMEM((_CHUNK, DIM), jnp.float32),
            pltpu.VMEM((_CHUNK, DIM), jnp.float32),
            pltpu.VMEM((_CHUNK, _W), jnp.int32),
            pltpu.VMEM((_CHUNK, _W), jnp.int32),
            pltpu.VMEM((_CHUNK, _W), jnp.int32),
            pltpu.SemaphoreType.DMA,
            pltpu.SemaphoreType.DMA,
            pltpu.SemaphoreType.DMA,
            pltpu.SemaphoreType.DMA,
            pltpu.SemaphoreType.DMA,
            pltpu.SemaphoreType.DMA,
        ],
    )
    def gather_kernel(ids_hbm, table_hbm, out_hbm, ids_v,
                      buf0, buf1, buf2, ob0, ob1, ob2,
                      gs0, gs1, gs2, os0, os1, os2):
        wid = lax.axis_index("s") * _NC + lax.axis_index("c")
        base = wid * per_w

        pltpu.sync_copy(ids_hbm.at[pl.ds(base, per_w)], ids_v)

        bufs = (buf0, buf1, buf2)
        obufs = (ob0, ob1, ob2)
        gsem = (gs0, gs1, gs2)
        osem = (os0, os1, os2)

        def start_gather(c, bi):
            pltpu.async_copy(
                table_hbm.at[ids_v.at[pl.ds(c * _CHUNK, _CHUNK)]],
                bufs[bi], gsem[bi])

        def wait_gather(bi):
            pltpu.make_async_copy(
                table_hbm.at[pl.ds(0, _CHUNK)], bufs[bi], gsem[bi]).wait()

        def start_out(c, bi):
            pltpu.async_copy(
                obufs[bi], out_hbm.at[pl.ds(base + c * _CHUNK, _CHUNK)],
                osem[bi])

        def wait_out(bi):
            pltpu.make_async_copy(
                obufs[bi], out_hbm.at[pl.ds(0, _CHUNK)], osem[bi]).wait()

        himask = jnp.int32(-65536)
        iota = lax.iota(jnp.int32, _LANES)
        idx_even = (iota & jnp.int32(7)) * jnp.int32(2)
        idx_odd = idx_even + jnp.int32(1)
        lo_half = iota < jnp.int32(8)

        def _deinterleave(a, b, idx):
            return jnp.where(lo_half, _lane_gather(a, idx),
                             _lane_gather(b, idx))

        def compact_chunk(bi):
            buf = bufs[bi]
            obuf = obufs[bi]

            def row(t, carry):
                for j in range(_W // _LANES):
                    a = buf[t, pl.ds(32 * j, _LANES)]
                    b = buf[t, pl.ds(32 * j + _LANES, _LANES)]
                    ia = lax.bitcast_convert_type(
                        _deinterleave(a, b, idx_even), jnp.int32)
                    ib = lax.bitcast_convert_type(
                        _deinterleave(a, b, idx_odd), jnp.int32)
                    obuf[t, pl.ds(j * _LANES, _LANES)] = (
                        lax.shift_right_logical(ia, jnp.int32(16))
                        | (ib & himask))
                return carry

            lax.fori_loop(0, _CHUNK, row, 0)

        start_gather(0, 0)
        start_gather(1, 1)

        def super_body(sp, carry):
            c0 = sp * 3
            wait_gather(0)
            compact_chunk(0)
            start_out(c0, 0)

            @pl.when(sp > 0)
            def _():
                wait_out(2)
            start_gather(c0 + 2, 2)

            wait_gather(1)
            compact_chunk(1)
            start_out(c0 + 1, 1)
            wait_out(0)
            start_gather(c0 + 3, 0)

            wait_gather(2)
            compact_chunk(2)
            start_out(c0 + 2, 2)
            wait_out(1)
            start_gather(c0 + 4, 1)
            return carry

        lax.fori_loop(0, n_super, super_body, 0)

        ce = n_super * 3
        wait_gather(0)
        compact_chunk(0)
        start_out(ce, 0)
        wait_gather(1)
        compact_chunk(1)
        start_out(ce + 1, 1)
        wait_out(2)
        wait_out(0)
        wait_out(1)

    return gather_kernel(ids_flat, word_table)


def _tc_body(g_ref, tt_ref, pos_ref, ty_ref, gb_ref, emb_ref):
    x = g_ref[...].astype(jnp.float32)
    pos = pos_ref[...]
    ty = ty_ref[...]
    gb = gb_ref[...]
    ttf = tt_ref[...]

    x = x + pos[None]
    t0 = ty[0][None, None, :]
    dt = (ty[1] - ty[0])[None, None, :]
    x = x + t0 + ttf[:, :, None] * dt

    mu = jnp.mean(x, axis=-1, keepdims=True)
    xc = x - mu
    var = jnp.mean(xc * xc, axis=-1, keepdims=True)
    y = xc * lax.rsqrt(var + EPS)
    y = y * gb[0][None, None, :] + gb[1][None, None, :]

    emb_ref[...] = y


def _tc_pos_body(pos_ref, out_ref):
    out_ref[...] = jnp.broadcast_to(pos_ref[...][None], out_ref.shape)


def kernel(input_ids, token_type_ids, word_table, pos_table, type_table, gamma, beta):
    b, s = input_ids.shape
    ids_flat = input_ids.reshape(-1).astype(jnp.int32)
    packed = _sc_gather_bf16(ids_flat, word_table)
    gathered = lax.bitcast_convert_type(packed, jnp.bfloat16).reshape(b, s, DIM)

    ttf = token_type_ids.astype(jnp.float32)
    pos_s = pos_table[:s]
    gb = jnp.stack([gamma, beta])

    br = 8
    grid = (b // br,)
    emb = pl.pallas_call(
        _tc_body,
        grid=grid,
        in_specs=[
            pl.BlockSpec((br, s, DIM), lambda i: (i, 0, 0)),
            pl.BlockSpec((br, s), lambda i: (i, 0)),
            pl.BlockSpec((s, DIM), lambda i: (0, 0)),
            pl.BlockSpec((2, DIM), lambda i: (0, 0)),
            pl.BlockSpec((2, DIM), lambda i: (0, 0)),
        ],
        out_specs=pl.BlockSpec((br, s, DIM), lambda i: (i, 0, 0)),
        out_shape=jax.ShapeDtypeStruct((b, s, DIM), jnp.float32),
    )(gathered, ttf, pos_s, type_table, gb)

    pos_out = pl.pallas_call(
        _tc_pos_body,
        grid=grid,
        in_specs=[pl.BlockSpec((s, DIM), lambda i: (0, 0))],
        out_specs=pl.BlockSpec((br, s, DIM), lambda i: (i, 0, 0)),
        out_shape=jax.ShapeDtypeStruct((b, s, DIM), jnp.float32),
    )(pos_s)

    return emb, pos_out


# speedup vs baseline: 3.3944x; 3.3944x over previous
import functools

import jax
import jax.numpy as jnp
from jax import lax
from jax.experimental import pallas as pl
from jax.experimental.pallas import tpu as pltpu
from jax.experimental.pallas import tpu_sc as plsc

DIM = 128
EPS = 1e-07

_NC = 2
_NS = 16
_NW = _NC * _NS
_CHUNK = 128


def _sc_gather(ids_flat, word_table):
    n_tok = ids_flat.shape[0]
    assert n_tok % (_NW * _CHUNK) == 0
    per_w = n_tok // _NW
    n_chunks = per_w // _CHUNK
    nb = 5
    assert n_chunks % nb == 0
    n_super = n_chunks // nb

    mesh = plsc.VectorSubcoreMesh(core_axis_name="c", subcore_axis_name="s")

    @functools.partial(
        pl.kernel,
        out_type=jax.ShapeDtypeStruct((n_tok, DIM), jnp.float32),
        mesh=mesh,
        scratch_types=(
            [pltpu.VMEM((per_w,), jnp.int32)]
            + [pltpu.VMEM((_CHUNK, DIM), jnp.float32)] * nb
            + [pltpu.SemaphoreType.DMA] * (2 * nb)
        ),
    )
    def gather_kernel(ids_hbm, table_hbm, out_hbm, ids_v, *rest):
        bufs = rest[:nb]
        gsem = rest[nb:2 * nb]
        osem = rest[2 * nb:]
        wid = lax.axis_index("s") * _NC + lax.axis_index("c")
        base = wid * per_w

        pltpu.sync_copy(ids_hbm.at[pl.ds(base, per_w)], ids_v)

        def start_gather(c, bi):
            pltpu.async_copy(
                table_hbm.at[ids_v.at[pl.ds(c * _CHUNK, _CHUNK)]],
                bufs[bi], gsem[bi])

        def wait_gather(bi):
            pltpu.make_async_copy(
                out_hbm.at[pl.ds(0, _CHUNK)], bufs[bi], gsem[bi]).wait()

        def start_out(c, bi):
            pltpu.async_copy(
                bufs[bi], out_hbm.at[pl.ds(base + c * _CHUNK, _CHUNK)],
                osem[bi])

        def wait_out(bi):
            pltpu.make_async_copy(
                bufs[bi], out_hbm.at[pl.ds(0, _CHUNK)], osem[bi]).wait()

        start_gather(0, 0)
        start_gather(1, 1)
        start_gather(2, 2)

        def super_body(sp, carry):
            for b in range(nb):
                c = sp * nb + b
                tb = (b + 3) % nb
                wait_gather(b)
                start_out(c, b)

                @pl.when(c + 3 < n_chunks)
                def _issue(tb=tb, c=c):
                    @pl.when(c >= 2)
                    def _drain():
                        wait_out(tb)
                    start_gather(c + 3, tb)
            return carry

        lax.fori_loop(0, n_super, super_body, 0)

        for b in range(nb):
            wait_out(b)

    return gather_kernel(ids_flat, word_table)


def _tc_body(g_ref, tt_ref, pos_ref, ty_ref, gb_ref, emb_ref):
    x = g_ref[...]
    pos = pos_ref[...]
    ty = ty_ref[...]
    gb = gb_ref[...]
    ttf = tt_ref[...]

    x = x + pos[None]
    t0 = ty[0][None, None, :]
    dt = (ty[1] - ty[0])[None, None, :]
    x = x + t0 + ttf[:, :, None] * dt

    mu = jnp.mean(x, axis=-1, keepdims=True)
    xc = x - mu
    var = jnp.mean(xc * xc, axis=-1, keepdims=True)
    y = xc * lax.rsqrt(var + EPS)
    y = y * gb[0][None, None, :] + gb[1][None, None, :]

    emb_ref[...] = y


def _tc_pos_body(pos_ref, out_ref):
    out_ref[...] = jnp.broadcast_to(pos_ref[...][None], out_ref.shape)


def kernel(input_ids, token_type_ids, word_table, pos_table, type_table, gamma, beta):
    b, s = input_ids.shape
    ids_flat = input_ids.reshape(-1).astype(jnp.int32)
    gathered = _sc_gather(ids_flat, word_table).reshape(b, s, DIM)

    ttf = token_type_ids.astype(jnp.float32)
    pos_s = pos_table[:s]
    gb = jnp.stack([gamma, beta])

    br = 8
    grid = (b // br,)
    emb = pl.pallas_call(
        _tc_body,
        grid=grid,
        in_specs=[
            pl.BlockSpec((br, s, DIM), lambda i: (i, 0, 0)),
            pl.BlockSpec((br, s), lambda i: (i, 0)),
            pl.BlockSpec((s, DIM), lambda i: (0, 0)),
            pl.BlockSpec((2, DIM), lambda i: (0, 0)),
            pl.BlockSpec((2, DIM), lambda i: (0, 0)),
        ],
        out_specs=pl.BlockSpec((br, s, DIM), lambda i: (i, 0, 0)),
        out_shape=jax.ShapeDtypeStruct((b, s, DIM), jnp.float32),
    )(gathered, ttf, pos_s, type_table, gb)

    pos_out = pl.pallas_call(
        _tc_pos_body,
        grid=grid,
        in_specs=[pl.BlockSpec((s, DIM), lambda i: (0, 0))],
        out_specs=pl.BlockSpec((br, s, DIM), lambda i: (i, 0, 0)),
        out_shape=jax.ShapeDtypeStruct((b, s, DIM), jnp.float32),
    )(pos_s)

    return emb, pos_out


# speedup vs baseline: 3.9862x; 1.1743x over previous
import functools

import jax
import jax.numpy as jnp
from jax import lax
from jax.experimental import pallas as pl
from jax.experimental.pallas import tpu as pltpu
from jax.experimental.pallas import tpu_sc as plsc

DIM = 128
EPS = 1e-07

_NC = 2
_NS = 16
_NW = _NC * _NS
_CHUNK = 128


def _sc_gather(ids_flat, word_table):
    n_tok = ids_flat.shape[0]
    assert n_tok % (_NW * _CHUNK) == 0
    per_w = n_tok // _NW
    n_chunks = per_w // _CHUNK
    nb = 5
    assert n_chunks % nb == 0
    n_super = n_chunks // nb

    mesh = plsc.VectorSubcoreMesh(core_axis_name="c", subcore_axis_name="s")

    @functools.partial(
        pl.kernel,
        out_type=jax.ShapeDtypeStruct((n_tok, DIM), jnp.float32),
        mesh=mesh,
        scratch_types=(
            [pltpu.VMEM((per_w,), jnp.int32)]
            + [pltpu.VMEM((_CHUNK, DIM), jnp.float32)] * nb
            + [pltpu.SemaphoreType.DMA] * (2 * nb)
        ),
    )
    def gather_kernel(ids_hbm, table_hbm, out_hbm, ids_v, *rest):
        bufs = rest[:nb]
        gsem = rest[nb:2 * nb]
        osem = rest[2 * nb:]
        wid = lax.axis_index("s") * _NC + lax.axis_index("c")
        base = wid * per_w

        pltpu.sync_copy(ids_hbm.at[pl.ds(base, per_w)], ids_v)

        def start_gather(c, bi):
            pltpu.async_copy(
                table_hbm.at[ids_v.at[pl.ds(c * _CHUNK, _CHUNK)]],
                bufs[bi], gsem[bi])

        def wait_gather(bi):
            pltpu.make_async_copy(
                out_hbm.at[pl.ds(0, _CHUNK)], bufs[bi], gsem[bi]).wait()

        def start_out(c, bi):
            pltpu.async_copy(
                bufs[bi], out_hbm.at[pl.ds(base + c * _CHUNK, _CHUNK)],
                osem[bi])

        def wait_out(bi):
            pltpu.make_async_copy(
                bufs[bi], out_hbm.at[pl.ds(0, _CHUNK)], osem[bi]).wait()

        start_gather(0, 0)
        start_gather(1, 1)
        start_gather(2, 2)

        def super_body(sp, carry):
            for b in range(nb):
                c = sp * nb + b
                tb = (b + 3) % nb
                wait_gather(b)
                start_out(c, b)

                @pl.when(c + 3 < n_chunks)
                def _issue(tb=tb, c=c):
                    @pl.when(c >= 2)
                    def _drain():
                        wait_out(tb)
                    start_gather(c + 3, tb)
            return carry

        lax.fori_loop(0, n_super, super_body, 0)

        for b in range(nb):
            wait_out(b)

    return gather_kernel(ids_flat, word_table)


def _tc_body(g_ref, tt_ref, pos_ref, ty_ref, gb_ref, emb_ref):
    x = g_ref[...]
    pos = pos_ref[...]
    ty = ty_ref[...]
    gb = gb_ref[...]
    ttf = tt_ref[...]

    x = x + pos[None]
    t0 = ty[0][None, None, :]
    dt = (ty[1] - ty[0])[None, None, :]
    x = x + t0 + ttf[:, :, None] * dt

    mu = jnp.mean(x, axis=-1, keepdims=True)
    xc = x - mu
    var = jnp.mean(xc * xc, axis=-1, keepdims=True)
    y = xc * lax.rsqrt(var + EPS)
    y = y * gb[0][None, None, :] + gb[1][None, None, :]

    emb_ref[...] = y


def _tc_pos_body(pos_ref, out_ref):
    out_ref[...] = jnp.broadcast_to(pos_ref[...][None], out_ref.shape)


def kernel(input_ids, token_type_ids, word_table, pos_table, type_table, gamma, beta):
    b, s = input_ids.shape
    ids_flat = input_ids.reshape(-1).astype(jnp.int32)
    gathered = _sc_gather(ids_flat, word_table).reshape(b, s, DIM)

    ttf = token_type_ids.astype(jnp.float32)
    pos_s = pos_table[:s]
    gb = jnp.stack([gamma, beta])

    br = 16
    grid = (b // br,)
    emb = pl.pallas_call(
        _tc_body,
        grid=grid,
        in_specs=[
            pl.BlockSpec((br, s, DIM), lambda i: (i, 0, 0)),
            pl.BlockSpec((br, s), lambda i: (i, 0)),
            pl.BlockSpec((s, DIM), lambda i: (0, 0)),
            pl.BlockSpec((2, DIM), lambda i: (0, 0)),
            pl.BlockSpec((2, DIM), lambda i: (0, 0)),
        ],
        out_specs=pl.BlockSpec((br, s, DIM), lambda i: (i, 0, 0)),
        out_shape=jax.ShapeDtypeStruct((b, s, DIM), jnp.float32),
    )(gathered, ttf, pos_s, type_table, gb)

    pos_out = pl.pallas_call(
        _tc_pos_body,
        grid=grid,
        in_specs=[pl.BlockSpec((s, DIM), lambda i: (0, 0))],
        out_specs=pl.BlockSpec((br, s, DIM), lambda i: (i, 0, 0)),
        out_shape=jax.ShapeDtypeStruct((b, s, DIM), jnp.float32),
    )(pos_s)

    return emb, pos_out


# speedup vs baseline: 4.2719x; 1.0717x over previous
import functools

import jax
import jax.numpy as jnp
from jax import lax
from jax.experimental import pallas as pl
from jax.experimental.pallas import tpu as pltpu
from jax.experimental.pallas import tpu_sc as plsc

DIM = 128
EPS = 1e-07

_NC = 2
_NS = 16
_NW = _NC * _NS
_CHUNK = 128


def _sc_gather(ids_flat, word_table):
    n_tok = ids_flat.shape[0]
    assert n_tok % (_NW * _CHUNK) == 0
    per_w = n_tok // _NW
    n_chunks = per_w // _CHUNK
    nb = 5
    assert n_chunks % nb == 0
    n_super = n_chunks // nb

    mesh = plsc.VectorSubcoreMesh(core_axis_name="c", subcore_axis_name="s")

    @functools.partial(
        pl.kernel,
        out_type=jax.ShapeDtypeStruct((n_tok, DIM), jnp.float32),
        mesh=mesh,
        scratch_types=(
            [pltpu.VMEM((per_w,), jnp.int32)]
            + [pltpu.VMEM((_CHUNK, DIM), jnp.float32)] * nb
            + [pltpu.SemaphoreType.DMA] * (2 * nb)
        ),
    )
    def gather_kernel(ids_hbm, table_hbm, out_hbm, ids_v, *rest):
        bufs = rest[:nb]
        gsem = rest[nb:2 * nb]
        osem = rest[2 * nb:]
        wid = lax.axis_index("s") * _NC + lax.axis_index("c")
        base = wid * per_w

        pltpu.sync_copy(ids_hbm.at[pl.ds(base, per_w)], ids_v)

        def start_gather(c, bi):
            pltpu.async_copy(
                table_hbm.at[ids_v.at[pl.ds(c * _CHUNK, _CHUNK)]],
                bufs[bi], gsem[bi])

        def wait_gather(bi):
            pltpu.make_async_copy(
                out_hbm.at[pl.ds(0, _CHUNK)], bufs[bi], gsem[bi]).wait()

        def start_out(c, bi):
            pltpu.async_copy(
                bufs[bi], out_hbm.at[pl.ds(base + c * _CHUNK, _CHUNK)],
                osem[bi])

        def wait_out(bi):
            pltpu.make_async_copy(
                bufs[bi], out_hbm.at[pl.ds(0, _CHUNK)], osem[bi]).wait()

        start_gather(0, 0)
        start_gather(1, 1)
        start_gather(2, 2)

        def super_body(sp, carry):
            for b in range(nb):
                c = sp * nb + b
                tb = (b + 3) % nb
                wait_gather(b)
                start_out(c, b)

                @pl.when(c + 3 < n_chunks)
                def _issue(tb=tb, c=c):
                    @pl.when(c >= 2)
                    def _drain():
                        wait_out(tb)
                    start_gather(c + 3, tb)
            return carry

        lax.fori_loop(0, n_super, super_body, 0)

        for b in range(nb):
            wait_out(b)

    return gather_kernel(ids_flat, word_table)


def _tc_body(g_ref, tt_ref, pos_ref, ty_ref, gb_ref, emb_ref):
    x = g_ref[...]
    pos = pos_ref[...]
    ty = ty_ref[...]
    gb = gb_ref[...]
    ttf = tt_ref[...]

    x = x + pos[None]
    t0 = ty[0][None, None, :]
    dt = (ty[1] - ty[0])[None, None, :]
    x = x + t0 + ttf[:, :, None] * dt

    mu = jnp.mean(x, axis=-1, keepdims=True)
    xc = x - mu
    var = jnp.mean(xc * xc, axis=-1, keepdims=True)
    y = xc * lax.rsqrt(var + EPS)
    y = y * gb[0][None, None, :] + gb[1][None, None, :]

    emb_ref[...] = y


def _tc_pos_body(pos_ref, out_ref):
    out_ref[...] = jnp.broadcast_to(pos_ref[...][None], out_ref.shape)


def kernel(input_ids, token_type_ids, word_table, pos_table, type_table, gamma, beta):
    b, s = input_ids.shape
    ids_flat = input_ids.reshape(-1).astype(jnp.int32)
    gathered = _sc_gather(ids_flat, word_table).reshape(b, s, DIM)

    ttf = token_type_ids.astype(jnp.float32)
    pos_s = pos_table[:s]
    gb = jnp.stack([gamma, beta])

    br = 32
    grid = (b // br,)
    emb = pl.pallas_call(
        _tc_body,
        grid=grid,
        in_specs=[
            pl.BlockSpec((br, s, DIM), lambda i: (i, 0, 0)),
            pl.BlockSpec((br, s), lambda i: (i, 0)),
            pl.BlockSpec((s, DIM), lambda i: (0, 0)),
            pl.BlockSpec((2, DIM), lambda i: (0, 0)),
            pl.BlockSpec((2, DIM), lambda i: (0, 0)),
        ],
        out_specs=pl.BlockSpec((br, s, DIM), lambda i: (i, 0, 0)),
        out_shape=jax.ShapeDtypeStruct((b, s, DIM), jnp.float32),
    )(gathered, ttf, pos_s, type_table, gb)

    pos_out = pl.pallas_call(
        _tc_pos_body,
        grid=grid,
        in_specs=[pl.BlockSpec((s, DIM), lambda i: (0, 0))],
        out_specs=pl.BlockSpec((br, s, DIM), lambda i: (i, 0, 0)),
        out_shape=jax.ShapeDtypeStruct((b, s, DIM), jnp.float32),
    )(pos_s)

    return emb, pos_out
